# baseline (speedup 1.0000x reference)
import functools

import jax
import jax.numpy as jnp
from jax import lax
from jax.experimental import pallas as pl
from jax.experimental.pallas import tpu as pltpu
from jax.experimental.pallas import tpu_sc as plsc

PATH_NUM = 64
N_TOKENS = 32768
NUM_CORES = 2
NUM_SUBCORES = 16
NUM_WORKERS = NUM_CORES * NUM_SUBCORES
ROWS_PER_WORKER = N_TOKENS // NUM_WORKERS
CHUNK_ROWS = 256
NUM_CHUNKS = ROWS_PER_WORKER // CHUNK_ROWS
LANES = 16
VPR = PATH_NUM // LANES
GROUP = 8

_NEG_INF = float("-inf")
_BIG_IDX = PATH_NUM


def _min_tree(xs):
    return jnp.minimum(jnp.minimum(xs[0], xs[1]), jnp.minimum(xs[2], xs[3]))


def _max_tree(xs):
    return jnp.maximum(jnp.maximum(xs[0], xs[1]), jnp.maximum(xs[2], xs[3]))


def _topk_body(score_hbm, out_hbm, vin0, vin1, vout0, vout1,
               isem0, isem1, osem0, osem1):
    wid = lax.axis_index("s") * NUM_CORES + lax.axis_index("c")
    iotas = [lax.iota(jnp.int32, LANES) + LANES * j for j in range(VPR)]
    one = jnp.full((LANES,), 1, jnp.int32)
    zero = jnp.full((LANES,), 0, jnp.int32)
    vins = [vin0, vin1]
    vouts = [vout0, vout1]
    isems = [isem0, isem1]
    osems = [osem0, osem1]

    def first_idx(eqs):
        cands = []
        for j in range(VPR):
            f = plsc.all_reduce_ffs(eqs[j])
            bad = jnp.logical_or(f > LANES - 1, f < 0)
            cands.append(jnp.where(bad, _BIG_IDX, f + LANES * j))
        return _min_tree(cands)

    def make_group_body(vin, vout):
      def group_body(i, carry):
        offs = [(i * GROUP + g) * PATH_NUM for g in range(GROUP)]
        vs = [[vin[pl.ds(offs[g] + LANES * j, LANES)] for j in range(VPR)]
              for g in range(GROUP)]
        s1 = [jnp.max(_max_tree(vs[g])) for g in range(GROUP)]
        eq1v = [[vs[g][j] == s1[g] for j in range(VPR)] for g in range(GROUP)]
        i1 = [first_idx(eq1v[g]) for g in range(GROUP)]
        eq1 = [[iotas[j] == i1[g] for j in range(VPR)] for g in range(GROUP)]
        vm = [[jnp.where(eq1[g][j], _NEG_INF, vs[g][j]) for j in range(VPR)]
              for g in range(GROUP)]
        s2 = [jnp.max(_max_tree(vm[g])) for g in range(GROUP)]
        eq2v = [[vm[g][j] == s2[g] for j in range(VPR)] for g in range(GROUP)]
        i2 = [first_idx(eq2v[g]) for g in range(GROUP)]
        for g in range(GROUP):
            for j in range(VPR):
                hit = jnp.logical_or(eq1[g][j], iotas[j] == i2[g])
                vout[pl.ds(offs[g] + LANES * j, LANES)] = \
                    jnp.where(hit, one, zero)
        return carry
      return group_body

    def hbm_slice(ch):
        base = (wid * ROWS_PER_WORKER + ch * CHUNK_ROWS) * PATH_NUM
        return pl.ds(base, CHUNK_ROWS * PATH_NUM)

    out_handles = [None, None]
    pltpu.async_copy(score_hbm.at[hbm_slice(0)], vins[0], isems[0])
    for ch in range(NUM_CHUNKS):
        cur = ch % 2
        if ch + 1 < NUM_CHUNKS:
            nxt = (ch + 1) % 2
            pltpu.async_copy(score_hbm.at[hbm_slice(ch + 1)], vins[nxt],
                             isems[nxt])
        pltpu.make_async_copy(score_hbm.at[hbm_slice(ch)], vins[cur],
                              isems[cur]).wait()
        if out_handles[cur] is not None:
            out_handles[cur].wait()
        lax.fori_loop(0, CHUNK_ROWS // GROUP,
                      make_group_body(vins[cur], vouts[cur]), 0)
        out_handles[cur] = pltpu.async_copy(
            vouts[cur], out_hbm.at[hbm_slice(ch)], osems[cur])
    for h in out_handles:
        if h is not None:
            h.wait()


@jax.jit
def kernel(score):
    mesh = plsc.VectorSubcoreMesh(
        core_axis_name="c", subcore_axis_name="s",
        num_cores=NUM_CORES, num_subcores=NUM_SUBCORES)
    run = pl.kernel(
        _topk_body,
        out_type=jax.ShapeDtypeStruct((N_TOKENS * PATH_NUM,), jnp.int32),
        mesh=mesh,
        scratch_types=[
            pltpu.VMEM((CHUNK_ROWS * PATH_NUM,), jnp.float32),
            pltpu.VMEM((CHUNK_ROWS * PATH_NUM,), jnp.float32),
            pltpu.VMEM((CHUNK_ROWS * PATH_NUM,), jnp.int32),
            pltpu.VMEM((CHUNK_ROWS * PATH_NUM,), jnp.int32),
            pltpu.SemaphoreType.DMA,
            pltpu.SemaphoreType.DMA,
            pltpu.SemaphoreType.DMA,
            pltpu.SemaphoreType.DMA,
        ],
        compiler_params=pltpu.CompilerParams(needs_layout_passes=False),
    )
    flat = run(score.reshape(-1))
    return flat.reshape(N_TOKENS, PATH_NUM)


# speedup vs baseline: 1.1364x; 1.1364x over previous
import functools

import jax
import jax.numpy as jnp
from jax import lax
from jax.experimental import pallas as pl
from jax.experimental.pallas import tpu as pltpu
from jax.experimental.pallas import tpu_sc as plsc

PATH_NUM = 64
N_TOKENS = 32768
NUM_CORES = 2
NUM_SUBCORES = 16
NUM_WORKERS = NUM_CORES * NUM_SUBCORES
ROWS_PER_WORKER = N_TOKENS // NUM_WORKERS
CHUNK_ROWS = 256
NUM_CHUNKS = ROWS_PER_WORKER // CHUNK_ROWS
LANES = 16
VPR = PATH_NUM // LANES
GROUP = 4

_NEG_INF = float("-inf")
_BIG_IDX = PATH_NUM


def _min_tree(xs):
    return jnp.minimum(jnp.minimum(xs[0], xs[1]), jnp.minimum(xs[2], xs[3]))


def _max_tree(xs):
    return jnp.maximum(jnp.maximum(xs[0], xs[1]), jnp.maximum(xs[2], xs[3]))


def _topk_body(score_hbm, out_hbm, vin0, vin1, vout0, vout1,
               isem0, isem1, osem0, osem1):
    wid = lax.axis_index("s") * NUM_CORES + lax.axis_index("c")
    iotas = [lax.iota(jnp.int32, LANES) + LANES * j for j in range(VPR)]
    one = jnp.full((LANES,), 1, jnp.int32)
    zero = jnp.full((LANES,), 0, jnp.int32)
    vins = [vin0, vin1]
    vouts = [vout0, vout1]
    isems = [isem0, isem1]
    osems = [osem0, osem1]

    def first_idx(eqs):
        cands = []
        for j in range(VPR):
            f = plsc.all_reduce_ffs(eqs[j])
            bad = jnp.logical_or(f > LANES - 1, f < 0)
            cands.append(jnp.where(bad, _BIG_IDX, f + LANES * j))
        return _min_tree(cands)

    def make_group_body(vin, vout):
      def group_body(i, carry):
        offs = [(i * GROUP + g) * PATH_NUM for g in range(GROUP)]
        vs = [[vin[pl.ds(offs[g] + LANES * j, LANES)] for j in range(VPR)]
              for g in range(GROUP)]
        s1 = [jnp.max(_max_tree(vs[g])) for g in range(GROUP)]
        eq1v = [[vs[g][j] == s1[g] for j in range(VPR)] for g in range(GROUP)]
        i1 = [first_idx(eq1v[g]) for g in range(GROUP)]
        eq1 = [[iotas[j] == i1[g] for j in range(VPR)] for g in range(GROUP)]
        vm = [[jnp.where(eq1[g][j], _NEG_INF, vs[g][j]) for j in range(VPR)]
              for g in range(GROUP)]
        s2 = [jnp.max(_max_tree(vm[g])) for g in range(GROUP)]
        eq2v = [[vm[g][j] == s2[g] for j in range(VPR)] for g in range(GROUP)]
        i2 = [first_idx(eq2v[g]) for g in range(GROUP)]
        for g in range(GROUP):
            for j in range(VPR):
                hit = jnp.logical_or(eq1[g][j], iotas[j] == i2[g])
                vout[pl.ds(offs[g] + LANES * j, LANES)] = \
                    jnp.where(hit, one, zero)
        return carry
      return group_body

    def hbm_slice(ch):
        base = (wid * ROWS_PER_WORKER + ch * CHUNK_ROWS) * PATH_NUM
        return pl.ds(base, CHUNK_ROWS * PATH_NUM)

    out_handles = [None, None]
    pltpu.async_copy(score_hbm.at[hbm_slice(0)], vins[0], isems[0])
    for ch in range(NUM_CHUNKS):
        cur = ch % 2
        if ch + 1 < NUM_CHUNKS:
            nxt = (ch + 1) % 2
            pltpu.async_copy(score_hbm.at[hbm_slice(ch + 1)], vins[nxt],
                             isems[nxt])
        pltpu.make_async_copy(score_hbm.at[hbm_slice(ch)], vins[cur],
                              isems[cur]).wait()
        if out_handles[cur] is not None:
            out_handles[cur].wait()
        lax.fori_loop(0, CHUNK_ROWS // GROUP,
                      make_group_body(vins[cur], vouts[cur]), 0)
        out_handles[cur] = pltpu.async_copy(
            vouts[cur], out_hbm.at[hbm_slice(ch)], osems[cur])
    for h in out_handles:
        if h is not None:
            h.wait()


@jax.jit
def kernel(score):
    mesh = plsc.VectorSubcoreMesh(
        core_axis_name="c", subcore_axis_name="s",
        num_cores=NUM_CORES, num_subcores=NUM_SUBCORES)
    run = pl.kernel(
        _topk_body,
        out_type=jax.ShapeDtypeStruct((N_TOKENS * PATH_NUM,), jnp.int32),
        mesh=mesh,
        scratch_types=[
            pltpu.VMEM((CHUNK_ROWS * PATH_NUM,), jnp.float32),
            pltpu.VMEM((CHUNK_ROWS * PATH_NUM,), jnp.float32),
            pltpu.VMEM((CHUNK_ROWS * PATH_NUM,), jnp.int32),
            pltpu.VMEM((CHUNK_ROWS * PATH_NUM,), jnp.int32),
            pltpu.SemaphoreType.DMA,
            pltpu.SemaphoreType.DMA,
            pltpu.SemaphoreType.DMA,
            pltpu.SemaphoreType.DMA,
        ],
        compiler_params=pltpu.CompilerParams(needs_layout_passes=False),
    )
    flat = run(score.reshape(-1))
    return flat.reshape(N_TOKENS, PATH_NUM)


# speedup vs baseline: 1.4826x; 1.3047x over previous
import functools

import jax
import jax.numpy as jnp
from jax import lax
from jax.experimental import pallas as pl
from jax.experimental.pallas import tpu as pltpu
from jax.experimental.pallas import tpu_sc as plsc

PATH_NUM = 64
N_TOKENS = 32768
NUM_CORES = 2
NUM_SUBCORES = 16
NUM_WORKERS = NUM_CORES * NUM_SUBCORES
ROWS_PER_WORKER = N_TOKENS // NUM_WORKERS
CHUNK_ROWS = 128
NUM_CHUNKS = ROWS_PER_WORKER // CHUNK_ROWS
LANES = 16
VPR = PATH_NUM // LANES
GROUP = 4

_NEG_INF = float("-inf")
_BIG_IDX = PATH_NUM


def _min_tree(xs):
    return jnp.minimum(jnp.minimum(xs[0], xs[1]), jnp.minimum(xs[2], xs[3]))


def _max_tree(xs):
    return jnp.maximum(jnp.maximum(xs[0], xs[1]), jnp.maximum(xs[2], xs[3]))


def _topk_body(score_hbm, out_hbm, vin0, vin1, vout0, vout1,
               isem0, isem1, osem0, osem1):
    wid = lax.axis_index("s") * NUM_CORES + lax.axis_index("c")
    iotas = [lax.iota(jnp.int32, LANES) + LANES * j for j in range(VPR)]
    one = jnp.full((LANES,), 1, jnp.int32)
    zero = jnp.full((LANES,), 0, jnp.int32)
    vins = [vin0, vin1]
    vouts = [vout0, vout1]
    isems = [isem0, isem1]
    osems = [osem0, osem1]

    def first_idx(eqs):
        cands = []
        for j in range(VPR):
            f = plsc.all_reduce_ffs(eqs[j])
            bad = jnp.logical_or(f > LANES - 1, f < 0)
            cands.append(jnp.where(bad, _BIG_IDX, f + LANES * j))
        return _min_tree(cands)

    def make_group_body(vin, vout):
      def group_body(i, carry):
        rows = [i * GROUP + g for g in range(GROUP)]
        vs = [[vin[rows[g], pl.ds(LANES * j, LANES)] for j in range(VPR)]
              for g in range(GROUP)]
        s1 = [jnp.max(_max_tree(vs[g])) for g in range(GROUP)]
        eq1v = [[vs[g][j] == s1[g] for j in range(VPR)] for g in range(GROUP)]
        i1 = [first_idx(eq1v[g]) for g in range(GROUP)]
        eq1 = [[iotas[j] == i1[g] for j in range(VPR)] for g in range(GROUP)]
        vm = [[jnp.where(eq1[g][j], _NEG_INF, vs[g][j]) for j in range(VPR)]
              for g in range(GROUP)]
        s2 = [jnp.max(_max_tree(vm[g])) for g in range(GROUP)]
        eq2v = [[vm[g][j] == s2[g] for j in range(VPR)] for g in range(GROUP)]
        i2 = [first_idx(eq2v[g]) for g in range(GROUP)]
        for g in range(GROUP):
            for j in range(VPR):
                hit = jnp.logical_or(eq1[g][j], iotas[j] == i2[g])
                vout[rows[g], pl.ds(LANES * j, LANES)] = \
                    jnp.where(hit, one, zero)
        return carry
      return group_body

    def hbm_slice(ch):
        base = wid * ROWS_PER_WORKER + ch * CHUNK_ROWS
        return pl.ds(base, CHUNK_ROWS)

    out_handles = [None, None]
    pltpu.async_copy(score_hbm.at[hbm_slice(0)], vins[0], isems[0])
    for ch in range(NUM_CHUNKS):
        cur = ch % 2
        if ch + 1 < NUM_CHUNKS:
            nxt = (ch + 1) % 2
            pltpu.async_copy(score_hbm.at[hbm_slice(ch + 1)], vins[nxt],
                             isems[nxt])
        pltpu.make_async_copy(score_hbm.at[hbm_slice(ch)], vins[cur],
                              isems[cur]).wait()
        if out_handles[cur] is not None:
            out_handles[cur].wait()
        lax.fori_loop(0, CHUNK_ROWS // GROUP,
                      make_group_body(vins[cur], vouts[cur]), 0)
        out_handles[cur] = pltpu.async_copy(
            vouts[cur], out_hbm.at[hbm_slice(ch)], osems[cur])
    for h in out_handles:
        if h is not None:
            h.wait()


@jax.jit
def kernel(score):
    mesh = plsc.VectorSubcoreMesh(
        core_axis_name="c", subcore_axis_name="s",
        num_cores=NUM_CORES, num_subcores=NUM_SUBCORES)
    run = pl.kernel(
        _topk_body,
        out_type=jax.ShapeDtypeStruct((N_TOKENS, PATH_NUM), jnp.int32),
        mesh=mesh,
        scratch_types=[
            pltpu.VMEM((CHUNK_ROWS, PATH_NUM), jnp.float32),
            pltpu.VMEM((CHUNK_ROWS, PATH_NUM), jnp.float32),
            pltpu.VMEM((CHUNK_ROWS, PATH_NUM), jnp.int32),
            pltpu.VMEM((CHUNK_ROWS, PATH_NUM), jnp.int32),
            pltpu.SemaphoreType.DMA,
            pltpu.SemaphoreType.DMA,
            pltpu.SemaphoreType.DMA,
            pltpu.SemaphoreType.DMA,
        ],
        compiler_params=pltpu.CompilerParams(needs_layout_passes=False),
    )
    return run(score)


# speedup vs baseline: 2.9499x; 1.9897x over previous
import jax
import jax.numpy as jnp
from jax import lax
from jax.experimental import pallas as pl
from jax.experimental.pallas import tpu as pltpu
from jax.experimental.pallas import tpu_sc as plsc

PATH_NUM = 64
N_TOKENS = 32768
NUM_CORES = 2
NUM_SUBCORES = 16
NUM_WORKERS = NUM_CORES * NUM_SUBCORES
LANES = 16
TGRID = 256
PGRID = 8
CG_PER_WORKER = TGRID // NUM_WORKERS
CGB = 2
NUM_CHUNKS = CG_PER_WORKER // CGB
GROUPS_PER_CHUNK = CGB * 128 // LANES
GB = 2

_NEG_INF = float("-inf")


def _topk_body(x_hbm, o_hbm, vin0, vin1, vout0, vout1,
               isem0, isem1, osem0, osem1):
    wid = lax.axis_index("s") * NUM_CORES + lax.axis_index("c")
    vins = [vin0, vin1]
    vouts = [vout0, vout1]
    isems = [isem0, isem1]
    osems = [osem0, osem1]
    lane = lax.iota(jnp.int32, LANES)
    zero16 = jnp.zeros((LANES,), jnp.int32)
    one16 = jnp.full((LANES,), 1, jnp.int32)
    ninf16 = jnp.full((LANES,), _NEG_INF, jnp.float32)

    def process_group(vin, vout, g):
        cgi = g // (128 // LANES)
        cc0 = (g % (128 // LANES)) * LANES
        m1, i1 = ninf16, zero16
        m2, i2 = ninf16, zero16
        for p in range(PATH_NUM):
            v = vin[p // 8, cgi, p % 8, pl.ds(cc0, LANES)]
            pc = jnp.full((LANES,), p, jnp.int32)
            gt1 = v > m1
            gt2 = v > m2
            m2n = jnp.where(gt2, v, m2)
            i2n = jnp.where(gt2, pc, i2)
            m2 = jnp.where(gt1, m1, m2n)
            i2 = jnp.where(gt1, i1, i2n)
            m1 = jnp.where(gt1, v, m1)
            i1 = jnp.where(gt1, pc, i1)
        cols = lane + cc0
        cg_s = zero16 + cgi
        plsc.store_scatter(vout, [i1 >> 3, cg_s, i1 & 7, cols], one16)
        plsc.store_scatter(vout, [i2 >> 3, cg_s, i2 & 7, cols], one16)

    def make_group_body(vin, vout):
        def group_body(i, carry):
            for gb in range(GB):
                process_group(vin, vout, i * GB + gb)
            return carry
        return group_body

    def zero_chunk(vout):
        def zb(z, carry):
            tg = z // (CGB * 8)
            rem = z % (CGB * 8)
            cgi = rem // 8
            r = rem % 8
            for q in range(128 // LANES):
                vout[tg, cgi, r, pl.ds(q * LANES, LANES)] = zero16
            return carry
        lax.fori_loop(0, PGRID * CGB * 8, zb, 0)

    def hbm_slice(ch):
        cg0 = wid * CG_PER_WORKER + ch * CGB
        return (slice(None), pl.ds(cg0, CGB), slice(None), slice(None))

    out_handles = [None, None]
    pltpu.async_copy(x_hbm.at[hbm_slice(0)], vins[0], isems[0])
    for ch in range(NUM_CHUNKS):
        cur = ch % 2
        if ch + 1 < NUM_CHUNKS:
            nxt = (ch + 1) % 2
            pltpu.async_copy(x_hbm.at[hbm_slice(ch + 1)], vins[nxt],
                             isems[nxt])
        if out_handles[cur] is not None:
            out_handles[cur].wait()
        zero_chunk(vouts[cur])
        pltpu.make_async_copy(x_hbm.at[hbm_slice(ch)], vins[cur],
                              isems[cur]).wait()
        lax.fori_loop(0, GROUPS_PER_CHUNK // GB,
                      make_group_body(vins[cur], vouts[cur]), 0)
        out_handles[cur] = pltpu.async_copy(
            vouts[cur], o_hbm.at[hbm_slice(ch)], osems[cur])
    for h in out_handles:
        if h is not None:
            h.wait()


@jax.jit
def kernel(score):
    mesh = plsc.VectorSubcoreMesh(
        core_axis_name="c", subcore_axis_name="s",
        num_cores=NUM_CORES, num_subcores=NUM_SUBCORES)
    run = pl.kernel(
        _topk_body,
        out_type=jax.ShapeDtypeStruct((PGRID, TGRID, 8, 128), jnp.int32),
        mesh=mesh,
        scratch_types=[
            pltpu.VMEM((PGRID, CGB, 8, 128), jnp.float32),
            pltpu.VMEM((PGRID, CGB, 8, 128), jnp.float32),
            pltpu.VMEM((PGRID, CGB, 8, 128), jnp.int32),
            pltpu.VMEM((PGRID, CGB, 8, 128), jnp.int32),
            pltpu.SemaphoreType.DMA,
            pltpu.SemaphoreType.DMA,
            pltpu.SemaphoreType.DMA,
            pltpu.SemaphoreType.DMA,
        ],
        compiler_params=pltpu.CompilerParams(needs_layout_passes=False),
    )
    x4 = jnp.transpose(jnp.reshape(score, (TGRID, 128, PGRID, 8)),
                       (2, 0, 3, 1))
    o4 = run(x4)
    return jnp.reshape(jnp.transpose(o4, (1, 3, 0, 2)),
                       (N_TOKENS, PATH_NUM))


# speedup vs baseline: 2.9586x; 1.0030x over previous
import jax
import jax.numpy as jnp
from jax import lax
from jax.experimental import pallas as pl
from jax.experimental.pallas import tpu as pltpu
from jax.experimental.pallas import tpu_sc as plsc

PATH_NUM = 64
N_TOKENS = 32768
NUM_CORES = 2
NUM_SUBCORES = 16
NUM_WORKERS = NUM_CORES * NUM_SUBCORES
LANES = 16
TGRID = 256
PGRID = 8
CG_PER_WORKER = TGRID // NUM_WORKERS
CGB = 2
NUM_CHUNKS = CG_PER_WORKER // CGB
GROUPS_PER_CHUNK = CGB * 128 // LANES
GB = 2

_NEG_INF = float("-inf")


def _topk_body(x_hbm, o_hbm, vin0, vin1, vout0, vout1,
               isem0, isem1, osem0, osem1):
    wid = lax.axis_index("s") * NUM_CORES + lax.axis_index("c")
    vins = [vin0, vin1]
    vouts = [vout0, vout1]
    isems = [isem0, isem1]
    osems = [osem0, osem1]
    lane = lax.iota(jnp.int32, LANES)
    zero16 = jnp.zeros((LANES,), jnp.int32)
    one16 = jnp.full((LANES,), 1, jnp.int32)
    ninf16 = jnp.full((LANES,), _NEG_INF, jnp.float32)

    def process_group(vin, vout, g):
        cgi = g // (128 // LANES)
        cc0 = (g % (128 // LANES)) * LANES
        m1, i1 = ninf16, zero16
        m2, i2 = ninf16, zero16
        for p in range(PATH_NUM):
            v = vin[p // 8, cgi, p % 8, pl.ds(cc0, LANES)]
            pc = jnp.full((LANES,), p, jnp.int32)
            gt1 = v > m1
            gt2 = v > m2
            m2n = jnp.where(gt2, v, m2)
            i2n = jnp.where(gt2, pc, i2)
            m2 = jnp.where(gt1, m1, m2n)
            i2 = jnp.where(gt1, i1, i2n)
            m1 = jnp.where(gt1, v, m1)
            i1 = jnp.where(gt1, pc, i1)
        cols = lane + cc0
        cg_s = zero16 + cgi
        plsc.store_scatter(vout, [i1 >> 3, cg_s, i1 & 7, cols], one16)
        plsc.store_scatter(vout, [i2 >> 3, cg_s, i2 & 7, cols], one16)

    def make_group_body(vin, vout):
        def group_body(i, carry):
            for gb in range(GB):
                process_group(vin, vout, i * GB + gb)
            return carry
        return group_body

    def zero_chunk(vout):
        def zb(z, carry):
            tg = z // (CGB * 8)
            rem = z % (CGB * 8)
            cgi = rem // 8
            r = rem % 8
            for q in range(128 // LANES):
                vout[tg, cgi, r, pl.ds(q * LANES, LANES)] = zero16
            return carry
        lax.fori_loop(0, PGRID * CGB * 8, zb, 0)

    def hbm_slice(ch):
        cg0 = wid * CG_PER_WORKER + ch * CGB
        return (slice(None), pl.ds(cg0, CGB), slice(None), slice(None))

    out_handles = [None, None]
    pltpu.async_copy(x_hbm.at[hbm_slice(0)], vins[0], isems[0])
    for ch in range(NUM_CHUNKS):
        cur = ch % 2
        if ch + 1 < NUM_CHUNKS:
            nxt = (ch + 1) % 2
            pltpu.async_copy(x_hbm.at[hbm_slice(ch + 1)], vins[nxt],
                             isems[nxt])
        if out_handles[cur] is not None:
            out_handles[cur].wait()
        zero_chunk(vouts[cur])
        pltpu.make_async_copy(x_hbm.at[hbm_slice(ch)], vins[cur],
                              isems[cur]).wait()
        lax.fori_loop(0, GROUPS_PER_CHUNK // GB,
                      make_group_body(vins[cur], vouts[cur]), 0)
        out_handles[cur] = pltpu.async_copy(
            vouts[cur], o_hbm.at[hbm_slice(ch)], osems[cur])
    for h in out_handles:
        if h is not None:
            h.wait()


@jax.jit
def kernel(score):
    mesh = plsc.VectorSubcoreMesh(
        core_axis_name="c", subcore_axis_name="s",
        num_cores=NUM_CORES, num_subcores=NUM_SUBCORES)
    run = pl.kernel(
        _topk_body,
        out_type=jax.ShapeDtypeStruct((PGRID, TGRID, 8, 128), jnp.int32),
        mesh=mesh,
        scratch_types=[
            pltpu.VMEM((PGRID, CGB, 8, 128), jnp.float32),
            pltpu.VMEM((PGRID, CGB, 8, 128), jnp.float32),
            pltpu.VMEM((PGRID, CGB, 8, 128), jnp.int32),
            pltpu.VMEM((PGRID, CGB, 8, 128), jnp.int32),
            pltpu.SemaphoreType.DMA,
            pltpu.SemaphoreType.DMA,
            pltpu.SemaphoreType.DMA,
            pltpu.SemaphoreType.DMA,
        ],
        compiler_params=pltpu.CompilerParams(needs_layout_passes=False, skip_device_barrier=True),
    )
    x4 = jnp.transpose(jnp.reshape(score, (TGRID, 128, PGRID, 8)),
                       (2, 0, 3, 1))
    o4 = run(x4)
    return jnp.reshape(jnp.transpose(o4, (1, 3, 0, 2)),
                       (N_TOKENS, PATH_NUM))
